# baseline (speedup 1.0000x reference)
import functools
import numpy as np
import jax
import jax.numpy as jnp
from jax import lax
from jax.experimental import pallas as pl
from jax.experimental.pallas import tpu as pltpu

B = 1024
A = 2
HW = 256
C = 26
NAGENTS = B * A
ROW = HW * C
R = 256
GRID = NAGENTS // R
BIG = 4096
NCHUNK = 4
CW = ROW // NCHUNK

_lane = np.arange(ROW)
_CELLS = jnp.array((_lane // C)[None, :], dtype=jnp.int32)
_KEYSRC = jnp.array(np.where(_lane % C == 0, _lane // C, BIG)[None, :],
                    dtype=jnp.int32)


def _fold26(x, op):
    n = x.shape[1]
    while n > C:
        n //= 2
        x = op(x[:, :n], x[:, n:2 * n])
    return x


def _body(obs_ref, rew_ref, cells_ref, keysrc_ref, out_ref):
    def group(g, _):
        r0 = pl.multiple_of(g * 8, 8)
        ch = [obs_ref[pl.ds(r0, 8), k * CW:(k + 1) * CW] for k in range(NCHUNK)]
        ks = [keysrc_ref[:, k * CW:(k + 1) * CW] for k in range(NCHUNK)]

        sacc = (ch[0] + ch[1]) + (ch[2] + ch[3])
        macc = jnp.maximum(jnp.maximum(ch[0], ch[1]),
                           jnp.maximum(ch[2], ch[3]))
        kp = [jnp.where(ch[k] > 0, ks[k], BIG) for k in range(NCHUNK)]
        kacc = jnp.minimum(jnp.minimum(kp[0], kp[1]),
                           jnp.minimum(kp[2], kp[3]))

        sums = _fold26(sacc, jnp.add)
        maxs = _fold26(macc, jnp.maximum)
        key26 = _fold26(kacc, jnp.minimum)
        key = key26[:, 0:1]

        found = key < BIG
        ax = key >> 4
        ay = key & 15
        interior = found & (ax >= 1) & (ax <= 14) & (ay >= 1) & (ay <= 14)
        loc = jnp.where(interior, (ax - 1) * 14 + (ay - 1), -1)

        s2, s3, s4, s5 = (sums[:, 2:3], sums[:, 3:4],
                          sums[:, 4:5], sums[:, 5:6])
        d = jnp.zeros((8, 1), jnp.int32)
        best = s2
        d = jnp.where(s3 > best, 1, d)
        best = jnp.maximum(best, s3)
        d = jnp.where(s4 > best, 2, d)
        best = jnp.maximum(best, s4)
        d = jnp.where(s5 > best, 3, d)
        dr = jnp.where(d == 0, -1, jnp.where(d == 1, 1, 0))
        dc = jnp.where(d == 2, 1, jnp.where(d == 3, -1, 0))
        axr = jnp.where(found, ax, -1)
        ayr = jnp.where(found, ay, -1)
        fx = axr + dr
        fy = ayr + dc
        fvalid = (fx >= 0) & (fx < 16) & (fy >= 0) & (fy < 16)
        facing = jnp.where(fvalid, fx * 16 + fy, -1)

        p = jnp.where(found, key, 255)
        ch2 = [obs_ref[pl.ds(r0, 8), k * CW:(k + 1) * CW]
               for k in range(NCHUNK)]
        cs = [cells_ref[:, k * CW:(k + 1) * CW] for k in range(NCHUNK)]
        pp = [jnp.where(cs[k] == p, ch2[k], -3.4e38) for k in range(NCHUNK)]
        pacc = jnp.maximum(jnp.maximum(pp[0], pp[1]),
                           jnp.maximum(pp[2], pp[3]))
        pv = _fold26(pacc, jnp.maximum)
        pot = pv[:, 10:11] > 0
        soup = pv[:, 21:22] > 0
        plate = pv[:, 22:23] > 0
        onion = pv[:, 23:24] > 0
        carrying = jnp.where(onion, 1, jnp.where(soup & (~pot), 3,
                   jnp.where(plate, 2, 0)))

        s16 = sums[:, 16:17]
        m20 = maxs[:, 20:21]
        m21 = maxs[:, 21:22]
        pot_state = jnp.where(m21 > 0., 10,
            jnp.where(m20 > 0.,
                jnp.where(m20 >= 17., 4, jnp.where(m20 >= 13., 5,
                jnp.where(m20 >= 9., 6, jnp.where(m20 >= 5., 7,
                jnp.where(m20 >= 2., 8, 9))))),
                jnp.where(s16 == 0., 0, jnp.where(s16 == 1., 1,
                jnp.where(s16 == 2., 2, 3)))))

        rw = rew_ref[pl.ds(r0, 8), :]
        goal = (rw[:, 0:1] >= 20.0) | (rw[:, 1:2] >= 20.0)

        out_ref[pl.ds(r0, 8), :] = jnp.concatenate([
            loc.astype(jnp.float32),
            facing.astype(jnp.float32),
            carrying.astype(jnp.float32),
            pot_state.astype(jnp.float32),
            goal.astype(jnp.float32),
        ], axis=1)
        return _

    lax.fori_loop(0, R // 8, group, None)


@functools.partial(jax.jit, static_argnames=("interpret",))
def _run(obs2, rew_pairs, interpret=False):
    return pl.pallas_call(
        _body,
        grid=(GRID,),
        in_specs=[
            pl.BlockSpec((R, ROW), lambda i: (i, 0)),
            pl.BlockSpec((R, A), lambda i: (i, 0)),
            pl.BlockSpec((1, ROW), lambda i: (0, 0)),
            pl.BlockSpec((1, ROW), lambda i: (0, 0)),
        ],
        out_specs=pl.BlockSpec((R, 5), lambda i: (i, 0)),
        out_shape=jax.ShapeDtypeStruct((NAGENTS, 5), jnp.float32),
        compiler_params=pltpu.CompilerParams(
            dimension_semantics=("arbitrary",)),
        interpret=interpret,
    )(obs2, rew_pairs, _CELLS, _KEYSRC)


def kernel(obs, rewards):
    obs2 = obs.reshape(NAGENTS, ROW)
    rew_pairs = jnp.broadcast_to(
        rewards.reshape(B, 1, A), (B, A, A)).reshape(NAGENTS, A)
    out = _run(obs2, rew_pairs)
    return out.reshape(B, A, 5)


# speedup vs baseline: 1.4059x; 1.4059x over previous
import functools
import jax
import jax.numpy as jnp
from jax import lax
from jax.experimental import pallas as pl
from jax.experimental.pallas import tpu as pltpu

B = 1024
A = 2
HW = 256
C = 26
NAGENTS = B * A
AG = 256
GRID = NAGENTS // AG
BIG = 4096


def _body(obs_ref, rew_ref, out_ref):
    cells = lax.broadcasted_iota(jnp.int32, (HW, 1), 0)

    pos = obs_ref[0]
    key = jnp.min(jnp.where(pos > 0, cells, BIG), axis=0)

    found = key < BIG
    ax = key >> 4
    ay = key & 15
    interior = found & (ax >= 1) & (ax <= 14) & (ay >= 1) & (ay <= 14)
    loc = jnp.where(interior, (ax - 1) * 14 + (ay - 1), -1)

    s2 = jnp.sum(obs_ref[2], axis=0)
    s3 = jnp.sum(obs_ref[3], axis=0)
    s4 = jnp.sum(obs_ref[4], axis=0)
    s5 = jnp.sum(obs_ref[5], axis=0)
    d = jnp.zeros((AG,), jnp.int32)
    best = s2
    d = jnp.where(s3 > best, 1, d)
    best = jnp.maximum(best, s3)
    d = jnp.where(s4 > best, 2, d)
    best = jnp.maximum(best, s4)
    d = jnp.where(s5 > best, 3, d)
    dr = jnp.where(d == 0, -1, jnp.where(d == 1, 1, 0))
    dc = jnp.where(d == 2, 1, jnp.where(d == 3, -1, 0))
    axr = jnp.where(found, ax, -1)
    ayr = jnp.where(found, ay, -1)
    fx = axr + dr
    fy = ayr + dc
    fvalid = (fx >= 0) & (fx < 16) & (fy >= 0) & (fy < 16)
    facing = jnp.where(fvalid, fx * 16 + fy, -1)

    p = jnp.where(found, key, 255)[None, :]
    onehot = cells == p
    pot = jnp.max(jnp.where(onehot, obs_ref[10], -3.4e38), axis=0) > 0
    soup = jnp.max(jnp.where(onehot, obs_ref[21], -3.4e38), axis=0) > 0
    plate = jnp.max(jnp.where(onehot, obs_ref[22], -3.4e38), axis=0) > 0
    onion = jnp.max(jnp.where(onehot, obs_ref[23], -3.4e38), axis=0) > 0
    carrying = jnp.where(onion, 1, jnp.where(soup & (~pot), 3,
               jnp.where(plate, 2, 0)))

    s16 = jnp.sum(obs_ref[16], axis=0)
    m20 = jnp.max(obs_ref[20], axis=0)
    m21 = jnp.max(obs_ref[21], axis=0)
    pot_state = jnp.where(m21 > 0., 10,
        jnp.where(m20 > 0.,
            jnp.where(m20 >= 17., 4, jnp.where(m20 >= 13., 5, jnp.where(m20 >= 9., 6,
            jnp.where(m20 >= 5., 7, jnp.where(m20 >= 2., 8, 9))))),
            jnp.where(s16 == 0., 0, jnp.where(s16 == 1., 1,
            jnp.where(s16 == 2., 2, 3)))))

    goal = (rew_ref[0] >= 20.0) | (rew_ref[1] >= 20.0)

    zf = jnp.zeros((AG,), jnp.float32)
    out_ref[...] = jnp.stack([
        loc.astype(jnp.float32),
        facing.astype(jnp.float32),
        carrying.astype(jnp.float32),
        pot_state.astype(jnp.float32),
        goal.astype(jnp.float32),
        zf, zf, zf,
    ], axis=0)


@functools.partial(jax.jit, static_argnames=("interpret",))
def _run(obs_t, rew_t, interpret=False):
    return pl.pallas_call(
        _body,
        grid=(GRID,),
        in_specs=[
            pl.BlockSpec((C, HW, AG), lambda i: (0, 0, i)),
            pl.BlockSpec((8, AG), lambda i: (0, i)),
        ],
        out_specs=pl.BlockSpec((8, AG), lambda i: (0, i)),
        out_shape=jax.ShapeDtypeStruct((8, NAGENTS), jnp.float32),
        compiler_params=pltpu.CompilerParams(
            dimension_semantics=("arbitrary",)),
        interpret=interpret,
    )(obs_t, rew_t)


def kernel(obs, rewards):
    obs_t = jnp.transpose(obs.reshape(NAGENTS, HW, C), (2, 1, 0))
    rew_t = jnp.broadcast_to(
        rewards.reshape(B, 1, A), (B, A, A)).reshape(NAGENTS, A).T
    rew8 = jnp.concatenate(
        [rew_t, jnp.zeros((8 - A, NAGENTS), jnp.float32)], axis=0)
    out = _run(obs_t, rew8)
    return out[:5].T.reshape(B, A, 5)


# speedup vs baseline: 2.1603x; 1.5366x over previous
import functools
import jax
import jax.numpy as jnp
from jax import lax
from jax.experimental import pallas as pl
from jax.experimental.pallas import tpu as pltpu

B = 1024
A = 2
HW = 256
C = 26
NAGENTS = B * A
R = 64
GRID = NAGENTS // R
BIG = 4096


def _body(obs_ref, rew_ref, out_ref):
    blk = obs_ref[...]
    blk_i = lax.bitcast_convert_type(blk, jnp.int32)
    cells = lax.broadcasted_iota(jnp.int32, (1, HW, 1), 1)

    sums = jnp.sum(blk, axis=1)
    maxs = jnp.max(blk_i, axis=1)
    pos = blk_i[:, :, 0:1]
    key = jnp.min(jnp.where(pos > 0, cells, BIG), axis=(1, 2))

    found = key < BIG
    ax = key >> 4
    ay = key & 15
    interior = found & (ax >= 1) & (ax <= 14) & (ay >= 1) & (ay <= 14)
    loc = jnp.where(interior, (ax - 1) * 14 + (ay - 1), -1)

    s2, s3, s4, s5 = sums[:, 2], sums[:, 3], sums[:, 4], sums[:, 5]
    d = jnp.zeros((R,), jnp.int32)
    best = s2
    d = jnp.where(s3 > best, 1, d)
    best = jnp.maximum(best, s3)
    d = jnp.where(s4 > best, 2, d)
    best = jnp.maximum(best, s4)
    d = jnp.where(s5 > best, 3, d)
    dr = jnp.where(d == 0, -1, jnp.where(d == 1, 1, 0))
    dc = jnp.where(d == 2, 1, jnp.where(d == 3, -1, 0))
    axr = jnp.where(found, ax, -1)
    ayr = jnp.where(found, ay, -1)
    fx = axr + dr
    fy = ayr + dc
    fvalid = (fx >= 0) & (fx < 16) & (fy >= 0) & (fy < 16)
    facing = jnp.where(fvalid, fx * 16 + fy, -1)

    p = jnp.where(found, key, 255)
    onehot = cells == p[:, None, None]
    pv = jnp.max(jnp.where(onehot, blk_i, jnp.int32(-2**31)), axis=1)
    pot = pv[:, 10] > 0
    soup = pv[:, 21] > 0
    plate = pv[:, 22] > 0
    onion = pv[:, 23] > 0
    carrying = jnp.where(onion, 1, jnp.where(soup & (~pot), 3,
               jnp.where(plate, 2, 0)))

    s16 = sums[:, 16]
    m20 = maxs[:, 20]
    m21 = maxs[:, 21]
    T17, T13, T9, T5, T2 = (0x41880000, 0x41500000, 0x41100000,
                            0x40A00000, 0x40000000)
    pot_state = jnp.where(m21 > 0, 10,
        jnp.where(m20 > 0,
            jnp.where(m20 >= T17, 4, jnp.where(m20 >= T13, 5, jnp.where(m20 >= T9, 6,
            jnp.where(m20 >= T5, 7, jnp.where(m20 >= T2, 8, 9))))),
            jnp.where(s16 == 0., 0, jnp.where(s16 == 1., 1,
            jnp.where(s16 == 2., 2, 3)))))

    rew = rew_ref[...]
    goal = (rew[:, 0] >= 20.0) | (rew[:, 1] >= 20.0)

    out_ref[...] = jnp.stack([
        loc.astype(jnp.float32),
        facing.astype(jnp.float32),
        carrying.astype(jnp.float32),
        pot_state.astype(jnp.float32),
        goal.astype(jnp.float32),
    ], axis=1)


@functools.partial(jax.jit, static_argnames=("interpret",))
def _run(obs3, rew2, interpret=False):
    return pl.pallas_call(
        _body,
        grid=(GRID,),
        in_specs=[
            pl.BlockSpec((R, HW, C), lambda i: (i, 0, 0)),
            pl.BlockSpec((R, A), lambda i: (i, 0)),
        ],
        out_specs=pl.BlockSpec((R, 5), lambda i: (i, 0)),
        out_shape=jax.ShapeDtypeStruct((NAGENTS, 5), jnp.float32),
        compiler_params=pltpu.CompilerParams(
            dimension_semantics=("arbitrary",)),
        interpret=interpret,
    )(obs3, rew2)


def kernel(obs, rewards):
    obs3 = obs.reshape(NAGENTS, HW, C)
    rew_pairs = jnp.broadcast_to(
        rewards.reshape(B, 1, A), (B, A, A)).reshape(NAGENTS, A)
    out = _run(obs3, rew_pairs)
    return out.reshape(B, A, 5)


# speedup vs baseline: 2.4005x; 1.1112x over previous
import functools
import jax
import jax.numpy as jnp
from jax import lax
from jax.experimental import pallas as pl
from jax.experimental.pallas import tpu as pltpu

B = 1024
A = 2
HW = 256
C = 26
NAGENTS = B * A
R = 64
GRID = NAGENTS // R
BIG = 4096


def _body(obs_ref, rew_ref, out_ref):
    blk = obs_ref[...]
    cells = lax.broadcasted_iota(jnp.int32, (1, HW, 1), 1)

    sums = jnp.sum(blk, axis=1)
    m = jnp.maximum(blk[:, :128], blk[:, 128:])
    m = jnp.maximum(m[:, :64], m[:, 64:])
    m = jnp.maximum(m[:, :32], m[:, 32:])
    m = jnp.maximum(m[:, :16], m[:, 16:])
    maxs = jnp.max(m, axis=1)
    pos = blk[:, :, 0:1]
    kf = jnp.where(pos > 0, cells, BIG)
    k2 = jnp.minimum(kf[:, :128], kf[:, 128:])
    k2 = jnp.minimum(k2[:, :64], k2[:, 64:])
    k2 = jnp.minimum(k2[:, :32], k2[:, 32:])
    k2 = jnp.minimum(k2[:, :16], k2[:, 16:])
    key = jnp.min(k2, axis=(1, 2))

    found = key < BIG
    ax = key >> 4
    ay = key & 15
    interior = found & (ax >= 1) & (ax <= 14) & (ay >= 1) & (ay <= 14)
    loc = jnp.where(interior, (ax - 1) * 14 + (ay - 1), -1)

    s2, s3, s4, s5 = sums[:, 2], sums[:, 3], sums[:, 4], sums[:, 5]
    d = jnp.zeros((R,), jnp.int32)
    best = s2
    d = jnp.where(s3 > best, 1, d)
    best = jnp.maximum(best, s3)
    d = jnp.where(s4 > best, 2, d)
    best = jnp.maximum(best, s4)
    d = jnp.where(s5 > best, 3, d)
    dr = jnp.where(d == 0, -1, jnp.where(d == 1, 1, 0))
    dc = jnp.where(d == 2, 1, jnp.where(d == 3, -1, 0))
    axr = jnp.where(found, ax, -1)
    ayr = jnp.where(found, ay, -1)
    fx = axr + dr
    fy = ayr + dc
    fvalid = (fx >= 0) & (fx < 16) & (fy >= 0) & (fy < 16)
    facing = jnp.where(fvalid, fx * 16 + fy, -1)

    p = jnp.where(found, key, 255)
    onehot = cells == p[:, None, None]
    pf = jnp.where(onehot, blk, -3.4e38)
    p2 = jnp.maximum(pf[:, :128], pf[:, 128:])
    p2 = jnp.maximum(p2[:, :64], p2[:, 64:])
    p2 = jnp.maximum(p2[:, :32], p2[:, 32:])
    p2 = jnp.maximum(p2[:, :16], p2[:, 16:])
    pv = jnp.max(p2, axis=1)
    pot = pv[:, 10] > 0
    soup = pv[:, 21] > 0
    plate = pv[:, 22] > 0
    onion = pv[:, 23] > 0
    carrying = jnp.where(onion, 1, jnp.where(soup & (~pot), 3,
               jnp.where(plate, 2, 0)))

    s16 = sums[:, 16]
    m20 = maxs[:, 20]
    m21 = maxs[:, 21]
    pot_state = jnp.where(m21 > 0., 10,
        jnp.where(m20 > 0.,
            jnp.where(m20 >= 17., 4, jnp.where(m20 >= 13., 5, jnp.where(m20 >= 9., 6,
            jnp.where(m20 >= 5., 7, jnp.where(m20 >= 2., 8, 9))))),
            jnp.where(s16 == 0., 0, jnp.where(s16 == 1., 1,
            jnp.where(s16 == 2., 2, 3)))))

    rew = rew_ref[...]
    goal = (rew[:, 0] >= 20.0) | (rew[:, 1] >= 20.0)

    out_ref[...] = jnp.stack([
        loc.astype(jnp.float32),
        facing.astype(jnp.float32),
        carrying.astype(jnp.float32),
        pot_state.astype(jnp.float32),
        goal.astype(jnp.float32),
    ], axis=1)


@functools.partial(jax.jit, static_argnames=("interpret",))
def _run(obs3, rew2, interpret=False):
    return pl.pallas_call(
        _body,
        grid=(GRID,),
        in_specs=[
            pl.BlockSpec((R, HW, C), lambda i: (i, 0, 0)),
            pl.BlockSpec((R, A), lambda i: (i, 0)),
        ],
        out_specs=pl.BlockSpec((R, 5), lambda i: (i, 0)),
        out_shape=jax.ShapeDtypeStruct((NAGENTS, 5), jnp.float32),
        compiler_params=pltpu.CompilerParams(
            dimension_semantics=("arbitrary",)),
        interpret=interpret,
    )(obs3, rew2)


def kernel(obs, rewards):
    obs3 = obs.reshape(NAGENTS, HW, C)
    rew_pairs = jnp.broadcast_to(
        rewards.reshape(B, 1, A), (B, A, A)).reshape(NAGENTS, A)
    out = _run(obs3, rew_pairs)
    return out.reshape(B, A, 5)
